# baseline (speedup 1.0000x reference)
import functools

import jax
import jax.numpy as jnp
from jax import lax
from jax.experimental import pallas as pl
from jax.experimental.pallas import tpu as pltpu
from jax.experimental.pallas import tpu_sc as plsc

N = 1024
E = 4096
D_IN = 128
D_OUT = 64
E_IN = 4
E_OUT = 16

NEG = -1e30

_NC = 2
_NS = 16
_NW = _NC * _NS
_EPW = E // _NW


def _lrelu(x):
    return jnp.where(x >= 0, x, 0.01 * x)


def _row_softmax(s, mask):
    s = jnp.where(mask, s, NEG)
    m = jnp.max(s, axis=-1, keepdims=True)
    e = jnp.where(mask, jnp.exp(s - m), 0.0)
    return e / jnp.maximum(jnp.sum(e, axis=-1, keepdims=True), 1e-30)


_JPR = 128 // E_IN


def _sc_gather_body(ef_rows, srcr, dstr, nfm,
                    ef_out, nfs_out, nfd_out,
                    src_v, dst_v, row_v, buf_v, nfs_v, nfd_v, sem):
    wid = lax.axis_index("s") * _NC + lax.axis_index("c")
    base = wid * _EPW
    pltpu.sync_copy(srcr.at[pl.ds(base, _EPW)], src_v)
    pltpu.sync_copy(dstr.at[pl.ds(base, _EPW)], dst_v)
    for i in range(_EPW // 16):
        sl = pl.ds(i * 16, 16)
        row_v[sl] = src_v[sl] * _JPR + (dst_v[sl] >> 5)
    c1 = pltpu.async_copy(ef_rows.at[row_v], buf_v, sem)
    c2 = pltpu.async_copy(nfm.at[src_v], nfs_v, sem)
    c3 = pltpu.async_copy(nfm.at[dst_v], nfd_v, sem)
    c1.wait()
    c2.wait()
    c3.wait()
    pltpu.sync_copy(buf_v, ef_out.at[pl.ds(base, _EPW)])
    pltpu.sync_copy(nfs_v, nfs_out.at[pl.ds(base, _EPW)])
    pltpu.sync_copy(nfd_v, nfd_out.at[pl.ds(base, _EPW)])


def _sc_gather(ef_rows, src, dst, nfm):
    mesh = plsc.VectorSubcoreMesh(core_axis_name="c", subcore_axis_name="s")
    f = pl.kernel(
        _sc_gather_body,
        mesh=mesh,
        out_type=[
            jax.ShapeDtypeStruct((E, _JPR * E_IN), jnp.float32),
            jax.ShapeDtypeStruct((E, D_IN), jnp.float32),
            jax.ShapeDtypeStruct((E, D_IN), jnp.float32),
        ],
        scratch_types=[
            pltpu.VMEM((_EPW,), jnp.int32),
            pltpu.VMEM((_EPW,), jnp.int32),
            pltpu.VMEM((_EPW,), jnp.int32),
            pltpu.VMEM((_EPW, _JPR * E_IN), jnp.float32),
            pltpu.VMEM((_EPW, D_IN), jnp.float32),
            pltpu.VMEM((_EPW, D_IN), jnp.float32),
            pltpu.SemaphoreType.DMA,
        ],
    )
    return f(ef_rows, src, dst, nfm)


BLK = 256
NBLK = E // BLK

_CD1 = (((1,), (1,)), ((), ()))


def _tc_small_body(consts, nfm, adj, src2, dst2, rows128, dstcol, nfs, nfd,
                   wnT, wnb, wrep, web, atnq, atnk, atee, aten,
                   qtnn, qtne, qtef,
                   out64, xei_out, efl_out, p_out):
    atn_b = consts[0:1, 0:1]
    ate_b = consts[0:1, 1:2]
    qtn_b = consts[0:1, 2:3]

    nf = jnp.dot(nfm[...], wnT[...],
                 preferred_element_type=jnp.float32) + wnb[...]
    jj = lax.broadcasted_iota(jnp.int32, (E, D_IN), 1)
    sel = (jj >> 2) == (dstcol[...] & (_JPR - 1))
    prod = jnp.where(sel, rows128[...], 0.0)
    ef_line = jnp.dot(prod, wrep[...],
                      preferred_element_type=jnp.float32) + web[...]
    efl_out[...] = ef_line
    p_out[...] = lax.dot_general(qtef[...], ef_line, _CD1,
                                 preferred_element_type=jnp.float32)

    deg = jnp.sum(adj[...], axis=1, keepdims=True)
    qn = lax.dot_general(atnq[...], nf, _CD1,
                         preferred_element_type=jnp.float32)
    kn = lax.dot_general(nf, atnk[...], _CD1,
                         preferred_element_type=jnp.float32)
    s_nn = _lrelu(qn + kn + atn_b)
    colid = lax.broadcasted_iota(jnp.int32, (N, N), 1).astype(jnp.float32)
    attn_nn = _row_softmax(s_nn, colid < deg)
    out64[0:N, :] = jnp.maximum(
        jnp.dot(attn_nn, nf, preferred_element_type=jnp.float32), 0.0)

    a_row = lax.dot_general(atee[...], ef_line, _CD1,
                            preferred_element_type=jnp.float32)
    c_col = lax.dot_general(nf, aten[...], _CD1,
                            preferred_element_type=jnp.float32)
    srcr = src2[...]
    dstr = dst2[...]
    ch = 256
    for c in range(N // ch):
        ids = lax.broadcasted_iota(jnp.int32, (ch, 1), 0) + c * ch
        inc = (srcr == ids) | (dstr == ids)
        s_ne = _lrelu(a_row + c_col[c * ch:(c + 1) * ch, :] + ate_b)
        attn = _row_softmax(s_ne, inc)
        xei_out[c * ch:(c + 1) * ch, :] = jnp.maximum(
            jnp.dot(attn, ef_line, preferred_element_type=jnp.float32), 0.0)

    nfsrc = jnp.dot(nfs[...], wnT[...],
                    preferred_element_type=jnp.float32) + wnb[...]
    nfdst = jnp.dot(nfd[...], wnT[...],
                    preferred_element_type=jnp.float32) + wnb[...]
    sl_s = lax.dot_general(nfsrc, qtnn[...], _CD1,
                           preferred_element_type=jnp.float32)
    sl_d = lax.dot_general(nfdst, qtnn[...], _CD1,
                           preferred_element_type=jnp.float32)
    er = lax.dot_general(ef_line, qtne[...], _CD1,
                         preferred_element_type=jnp.float32)
    s0 = _lrelu(sl_s + er + qtn_b)
    s1 = _lrelu(sl_d + er + qtn_b)
    m = jnp.maximum(s0, s1)
    e0 = jnp.exp(s0 - m)
    e1 = jnp.exp(s1 - m)
    den = e0 + e1
    out64[N:N + E, :] = jnp.maximum(
        (e0 / den) * nfsrc + (e1 / den) * nfdst, 0.0)


def _tc_small(consts, nfm, adj, src2, dst2, rows128, dstcol, nfs, nfd,
              wnT, wnb, wrep, web, atnq, atnk, atee, aten, qtnn, qtne, qtef):
    full = lambda shape: pl.BlockSpec(shape, lambda: (0,) * len(shape))
    return pl.pallas_call(
        _tc_small_body,
        in_specs=[
            full((1, 8)),
            full((N, D_IN)),
            full((N, N)),
            full((1, E)),
            full((1, E)),
            full((E, D_IN)),
            full((E, 1)),
            full((E, D_IN)),
            full((E, D_IN)),
            full((D_IN, D_OUT)),
            full((1, D_OUT)),
            full((D_IN, E_OUT)),
            full((1, E_OUT)),
            full((1, D_OUT)),
            full((1, D_OUT)),
            full((1, E_OUT)),
            full((1, D_OUT)),
            full((1, D_OUT)),
            full((1, E_OUT)),
            full((1, E_OUT)),
        ],
        out_specs=[
            full((N + E, D_OUT)),
            full((N, E_OUT)),
            full((E, E_OUT)),
            full((1, E)),
        ],
        out_shape=[
            jax.ShapeDtypeStruct((N + E, D_OUT), jnp.float32),
            jax.ShapeDtypeStruct((N, E_OUT), jnp.float32),
            jax.ShapeDtypeStruct((E, E_OUT), jnp.float32),
            jax.ShapeDtypeStruct((1, E), jnp.float32),
        ],
    )(consts, nfm, adj, src2, dst2, rows128, dstcol, nfs, nfd,
      wnT, wnb, wrep, web, atnq, atnk, atee, aten, qtnn, qtne, qtef)


def _tc_ee_body(consts, efl, p_row, qtee, ladj, out):
    i = pl.program_id(0)
    qte_b = consts[0:1, 3:4]
    e_full = efl[...]
    qv = lax.dot_general(efl[pl.ds(i * BLK, BLK), :], qtee[...], _CD1,
                         preferred_element_type=jnp.float32)
    s_ee = _lrelu(p_row[...] + qv + qte_b)
    attn = _row_softmax(s_ee, ladj[...] > 0)
    out[...] = jnp.maximum(
        jnp.dot(attn, e_full, preferred_element_type=jnp.float32), 0.0)


def _tc_ee(consts, efl, p_row, qtee, ladj):
    full = lambda shape: pl.BlockSpec(shape, lambda i: (0,) * len(shape))
    return pl.pallas_call(
        _tc_ee_body,
        grid=(NBLK,),
        in_specs=[
            full((1, 8)),
            full((E, E_OUT)),
            full((1, E)),
            full((1, E_OUT)),
            pl.BlockSpec((BLK, E), lambda i: (i, 0)),
        ],
        out_specs=pl.BlockSpec((BLK, E_OUT), lambda i: (i, 0)),
        out_shape=jax.ShapeDtypeStruct((E, E_OUT), jnp.float32),
    )(consts, efl, p_row, qtee, ladj)


def kernel(node_features, edge_index, line_adj_matrix, adj_matrix,
           edge_features, Wn_w, Wn_b, We_w, We_b, atn_w, atn_b, ate_w, ate_b,
           qtn_w, qtn_b, qte_w, qte_b):
    src = edge_index[0].astype(jnp.int32)
    dst = edge_index[1].astype(jnp.int32)
    ef_guard = jnp.where(src[0] >= 0, edge_features, 0.0)
    ef_rows = ef_guard.reshape(N * N // _JPR, _JPR * E_IN)

    rows128, nfs_raw, nfd_raw = _sc_gather(ef_rows, src, dst, node_features)

    consts = jnp.concatenate([
        atn_b.reshape(1), ate_b.reshape(1), qtn_b.reshape(1),
        qte_b.reshape(1), jnp.zeros((4,), jnp.float32)]).reshape(1, 8)

    out64, x_ei, ef_line, p_row = _tc_small(
        consts, node_features, adj_matrix,
        src.reshape(1, E), dst.reshape(1, E),
        rows128, dst.reshape(E, 1), nfs_raw, nfd_raw,
        Wn_w.T, Wn_b.reshape(1, D_OUT),
        jnp.tile(We_w.T, (_JPR, 1)), We_b.reshape(1, E_OUT),
        atn_w[:, :D_OUT], atn_w[:, D_OUT:],
        ate_w[:, :E_OUT], ate_w[:, E_OUT:],
        qtn_w[:, :D_OUT], qtn_w[:, D_OUT:],
        qte_w[:, :E_OUT])

    e_ei = _tc_ee(consts, ef_line, p_row, qte_w[:, E_OUT:], line_adj_matrix)

    out16 = jnp.concatenate([x_ei, e_ei], axis=0)
    return jnp.concatenate([out64, out16], axis=1)


# speedup vs baseline: 13.8121x; 13.8121x over previous
import functools

import jax
import jax.numpy as jnp
from jax import lax
from jax.experimental import pallas as pl
from jax.experimental.pallas import tpu as pltpu
from jax.experimental.pallas import tpu_sc as plsc

N = 1024
E = 4096
D_IN = 128
D_OUT = 64
E_IN = 4
E_OUT = 16

NEG = -1e30

_NC = 2
_NS = 16
_NW = _NC * _NS
_EPW = E // _NW


def _lrelu(x):
    return jnp.where(x >= 0, x, 0.01 * x)


def _row_softmax(s, mask):
    s = jnp.where(mask, s, NEG)
    m = jnp.max(s, axis=-1, keepdims=True)
    e = jnp.where(mask, jnp.exp(s - m), 0.0)
    return e / jnp.maximum(jnp.sum(e, axis=-1, keepdims=True), 1e-30)


_NJB = N // 128


def _sc_gather_body(ef_rows, srcr, dstr, nfm,
                    ef0_out, ef1_out, ef2_out, ef3_out, nfs_out, nfd_out,
                    src_v, dst_v, row_v, b0_v, b1_v, b2_v, b3_v,
                    nfs_v, nfd_v, sem):
    wid = lax.axis_index("s") * _NC + lax.axis_index("c")
    base = wid * _EPW
    pltpu.sync_copy(srcr.at[pl.ds(base, _EPW)], src_v)
    pltpu.sync_copy(dstr.at[pl.ds(base, _EPW)], dst_v)
    for i in range(_EPW // 16):
        sl = pl.ds(i * 16, 16)
        row_v[sl] = src_v[sl] * (_NJB * E_IN) + ((dst_v[sl] >> 7) << 2)
    c2 = pltpu.async_copy(nfm.at[src_v], nfs_v, sem)
    c3 = pltpu.async_copy(nfm.at[dst_v], nfd_v, sem)
    bufs = (b0_v, b1_v, b2_v, b3_v)
    outs = (ef0_out, ef1_out, ef2_out, ef3_out)
    for k in range(E_IN):
        pltpu.async_copy(ef_rows.at[row_v], bufs[k], sem).wait()
        pltpu.sync_copy(bufs[k], outs[k].at[pl.ds(base, _EPW)])
        if k < E_IN - 1:
            for i in range(_EPW // 16):
                sl = pl.ds(i * 16, 16)
                row_v[sl] = row_v[sl] + 1
    c2.wait()
    c3.wait()
    pltpu.sync_copy(nfs_v, nfs_out.at[pl.ds(base, _EPW)])
    pltpu.sync_copy(nfd_v, nfd_out.at[pl.ds(base, _EPW)])


def _sc_gather(ef_rows, src, dst, nfm):
    mesh = plsc.VectorSubcoreMesh(core_axis_name="c", subcore_axis_name="s")
    f = pl.kernel(
        _sc_gather_body,
        mesh=mesh,
        out_type=[
            jax.ShapeDtypeStruct((E, 128), jnp.float32),
            jax.ShapeDtypeStruct((E, 128), jnp.float32),
            jax.ShapeDtypeStruct((E, 128), jnp.float32),
            jax.ShapeDtypeStruct((E, 128), jnp.float32),
            jax.ShapeDtypeStruct((E, D_IN), jnp.float32),
            jax.ShapeDtypeStruct((E, D_IN), jnp.float32),
        ],
        scratch_types=[
            pltpu.VMEM((_EPW,), jnp.int32),
            pltpu.VMEM((_EPW,), jnp.int32),
            pltpu.VMEM((_EPW,), jnp.int32),
            pltpu.VMEM((_EPW, 128), jnp.float32),
            pltpu.VMEM((_EPW, 128), jnp.float32),
            pltpu.VMEM((_EPW, 128), jnp.float32),
            pltpu.VMEM((_EPW, 128), jnp.float32),
            pltpu.VMEM((_EPW, D_IN), jnp.float32),
            pltpu.VMEM((_EPW, D_IN), jnp.float32),
            pltpu.SemaphoreType.DMA,
        ],
    )
    return f(ef_rows, src, dst, nfm)


BLK = 256
NBLK = E // BLK

_CD1 = (((1,), (1,)), ((), ()))


def _tc_small_body(consts, nfm, adj, src2, dst2, b0, b1, b2, b3, dstcol,
                   nfs, nfd, wnT, wnb, weflat, web, atnq, atnk, atee, aten,
                   qtnn, qtne, qtef,
                   out64, xei_out, efl_out, p_out):
    atn_b = consts[0:1, 0:1]
    ate_b = consts[0:1, 1:2]
    qtn_b = consts[0:1, 2:3]

    nf = jnp.dot(nfm[...], wnT[...],
                 preferred_element_type=jnp.float32) + wnb[...]
    jj = lax.broadcasted_iota(jnp.int32, (E, 128), 1)
    sel = jj == (dstcol[...] & 127)
    ef_line = web[...]
    for k, bk in enumerate((b0, b1, b2, b3)):
        v_k = jnp.sum(jnp.where(sel, bk[...], 0.0), axis=1, keepdims=True)
        ef_line = ef_line + v_k * weflat[0:1, k * E_OUT:(k + 1) * E_OUT]
    efl_out[...] = ef_line
    p_out[...] = lax.dot_general(qtef[...], ef_line, _CD1,
                                 preferred_element_type=jnp.float32)

    deg = jnp.sum(adj[...], axis=1, keepdims=True)
    qn = lax.dot_general(atnq[...], nf, _CD1,
                         preferred_element_type=jnp.float32)
    kn = lax.dot_general(nf, atnk[...], _CD1,
                         preferred_element_type=jnp.float32)
    s_nn = _lrelu(qn + kn + atn_b)
    colid = lax.broadcasted_iota(jnp.int32, (N, N), 1).astype(jnp.float32)
    attn_nn = _row_softmax(s_nn, colid < deg)
    out64[0:N, :] = jnp.maximum(
        jnp.dot(attn_nn, nf, preferred_element_type=jnp.float32), 0.0)

    a_row = lax.dot_general(atee[...], ef_line, _CD1,
                            preferred_element_type=jnp.float32)
    c_col = lax.dot_general(nf, aten[...], _CD1,
                            preferred_element_type=jnp.float32)
    srcr = src2[...]
    dstr = dst2[...]
    ch = 256
    for c in range(N // ch):
        ids = lax.broadcasted_iota(jnp.int32, (ch, 1), 0) + c * ch
        inc = (srcr == ids) | (dstr == ids)
        s_ne = _lrelu(a_row + c_col[c * ch:(c + 1) * ch, :] + ate_b)
        attn = _row_softmax(s_ne, inc)
        xei_out[c * ch:(c + 1) * ch, :] = jnp.maximum(
            jnp.dot(attn, ef_line, preferred_element_type=jnp.float32), 0.0)

    nfsrc = jnp.dot(nfs[...], wnT[...],
                    preferred_element_type=jnp.float32) + wnb[...]
    nfdst = jnp.dot(nfd[...], wnT[...],
                    preferred_element_type=jnp.float32) + wnb[...]
    sl_s = lax.dot_general(nfsrc, qtnn[...], _CD1,
                           preferred_element_type=jnp.float32)
    sl_d = lax.dot_general(nfdst, qtnn[...], _CD1,
                           preferred_element_type=jnp.float32)
    er = lax.dot_general(ef_line, qtne[...], _CD1,
                         preferred_element_type=jnp.float32)
    s0 = _lrelu(sl_s + er + qtn_b)
    s1 = _lrelu(sl_d + er + qtn_b)
    m = jnp.maximum(s0, s1)
    e0 = jnp.exp(s0 - m)
    e1 = jnp.exp(s1 - m)
    den = e0 + e1
    out64[N:N + E, :] = jnp.maximum(
        (e0 / den) * nfsrc + (e1 / den) * nfdst, 0.0)


def _tc_small(consts, nfm, adj, src2, dst2, b0, b1, b2, b3, dstcol, nfs, nfd,
              wnT, wnb, weflat, web, atnq, atnk, atee, aten, qtnn, qtne, qtef):
    full = lambda shape: pl.BlockSpec(shape, lambda: (0,) * len(shape))
    return pl.pallas_call(
        _tc_small_body,
        in_specs=[
            full((1, 8)),
            full((N, D_IN)),
            full((N, N)),
            full((1, E)),
            full((1, E)),
            full((E, 128)),
            full((E, 128)),
            full((E, 128)),
            full((E, 128)),
            full((E, 1)),
            full((E, D_IN)),
            full((E, D_IN)),
            full((D_IN, D_OUT)),
            full((1, D_OUT)),
            full((1, E_IN * E_OUT)),
            full((1, E_OUT)),
            full((1, D_OUT)),
            full((1, D_OUT)),
            full((1, E_OUT)),
            full((1, D_OUT)),
            full((1, D_OUT)),
            full((1, E_OUT)),
            full((1, E_OUT)),
        ],
        out_specs=[
            full((N + E, D_OUT)),
            full((N, E_OUT)),
            full((E, E_OUT)),
            full((1, E)),
        ],
        out_shape=[
            jax.ShapeDtypeStruct((N + E, D_OUT), jnp.float32),
            jax.ShapeDtypeStruct((N, E_OUT), jnp.float32),
            jax.ShapeDtypeStruct((E, E_OUT), jnp.float32),
            jax.ShapeDtypeStruct((1, E), jnp.float32),
        ],
    )(consts, nfm, adj, src2, dst2, b0, b1, b2, b3, dstcol, nfs, nfd,
      wnT, wnb, weflat, web, atnq, atnk, atee, aten, qtnn, qtne, qtef)


def _tc_ee_body(consts, efl, p_row, qtee, ladj, out):
    i = pl.program_id(0)
    qte_b = consts[0:1, 3:4]
    e_full = efl[...]
    qv = lax.dot_general(efl[pl.ds(i * BLK, BLK), :], qtee[...], _CD1,
                         preferred_element_type=jnp.float32)
    s_ee = _lrelu(p_row[...] + qv + qte_b)
    attn = _row_softmax(s_ee, ladj[...] > 0)
    out[...] = jnp.maximum(
        jnp.dot(attn, e_full, preferred_element_type=jnp.float32), 0.0)


def _tc_ee(consts, efl, p_row, qtee, ladj):
    full = lambda shape: pl.BlockSpec(shape, lambda i: (0,) * len(shape))
    return pl.pallas_call(
        _tc_ee_body,
        grid=(NBLK,),
        in_specs=[
            full((1, 8)),
            full((E, E_OUT)),
            full((1, E)),
            full((1, E_OUT)),
            pl.BlockSpec((BLK, E), lambda i: (i, 0)),
        ],
        out_specs=pl.BlockSpec((BLK, E_OUT), lambda i: (i, 0)),
        out_shape=jax.ShapeDtypeStruct((E, E_OUT), jnp.float32),
    )(consts, efl, p_row, qtee, ladj)


def kernel(node_features, edge_index, line_adj_matrix, adj_matrix,
           edge_features, Wn_w, Wn_b, We_w, We_b, atn_w, atn_b, ate_w, ate_b,
           qtn_w, qtn_b, qte_w, qte_b):
    src = edge_index[0].astype(jnp.int32)
    dst = edge_index[1].astype(jnp.int32)
    ef_rows = (edge_features.reshape(N, _NJB, 128, E_IN)
               .transpose(0, 1, 3, 2)
               .reshape(N * _NJB * E_IN, 128))

    b0, b1, b2, b3, nfs_raw, nfd_raw = _sc_gather(
        ef_rows, src, dst, node_features)

    consts = jnp.concatenate([
        atn_b.reshape(1), ate_b.reshape(1), qtn_b.reshape(1),
        qte_b.reshape(1), jnp.zeros((4,), jnp.float32)]).reshape(1, 8)

    out64, x_ei, ef_line, p_row = _tc_small(
        consts, node_features, adj_matrix,
        src.reshape(1, E), dst.reshape(1, E),
        b0, b1, b2, b3, dst.reshape(E, 1), nfs_raw, nfd_raw,
        Wn_w.T, Wn_b.reshape(1, D_OUT),
        We_w.T.reshape(1, E_IN * E_OUT), We_b.reshape(1, E_OUT),
        atn_w[:, :D_OUT], atn_w[:, D_OUT:],
        ate_w[:, :E_OUT], ate_w[:, E_OUT:],
        qtn_w[:, :D_OUT], qtn_w[:, D_OUT:],
        qte_w[:, :E_OUT])

    e_ei = _tc_ee(consts, ef_line, p_row, qte_w[:, E_OUT:], line_adj_matrix)

    out16 = jnp.concatenate([x_ei, e_ei], axis=0)
    return jnp.concatenate([out64, out16], axis=1)


# speedup vs baseline: 13.9551x; 1.0104x over previous
import functools

import jax
import jax.numpy as jnp
from jax import lax
from jax.experimental import pallas as pl
from jax.experimental.pallas import tpu as pltpu
from jax.experimental.pallas import tpu_sc as plsc

N = 1024
E = 4096
D_IN = 128
D_OUT = 64
E_IN = 4
E_OUT = 16

NEG = -1e30

_NC = 2
_NS = 16
_NW = _NC * _NS
_EPW = E // _NW


def _lrelu(x):
    return jnp.where(x >= 0, x, 0.01 * x)


def _row_softmax(s, mask):
    s = jnp.where(mask, s, NEG)
    m = jnp.max(s, axis=-1, keepdims=True)
    e = jnp.where(mask, jnp.exp(s - m), 0.0)
    return e / jnp.maximum(jnp.sum(e, axis=-1, keepdims=True), 1e-30)


_NJB = N // 128


def _sc_gather_body(ef_rows, srcr, dstr, nfm,
                    ef0_out, ef1_out, ef2_out, ef3_out, nfs_out, nfd_out,
                    src_v, dst_v, row_v, b0_v, b1_v, b2_v, b3_v,
                    nfs_v, nfd_v, sem):
    wid = lax.axis_index("s") * _NC + lax.axis_index("c")
    base = wid * _EPW
    pltpu.sync_copy(srcr.at[pl.ds(base, _EPW)], src_v)
    pltpu.sync_copy(dstr.at[pl.ds(base, _EPW)], dst_v)
    for i in range(_EPW // 16):
        sl = pl.ds(i * 16, 16)
        row_v[sl] = src_v[sl] * (_NJB * E_IN) + ((dst_v[sl] >> 7) << 2)
    c2 = pltpu.async_copy(nfm.at[src_v], nfs_v, sem)
    c3 = pltpu.async_copy(nfm.at[dst_v], nfd_v, sem)
    bufs = (b0_v, b1_v, b2_v, b3_v)
    outs = (ef0_out, ef1_out, ef2_out, ef3_out)
    for k in range(E_IN):
        pltpu.async_copy(ef_rows.at[row_v], bufs[k], sem).wait()
        pltpu.sync_copy(bufs[k], outs[k].at[pl.ds(base, _EPW)])
        if k < E_IN - 1:
            for i in range(_EPW // 16):
                sl = pl.ds(i * 16, 16)
                row_v[sl] = row_v[sl] + 1
    c2.wait()
    c3.wait()
    pltpu.sync_copy(nfs_v, nfs_out.at[pl.ds(base, _EPW)])
    pltpu.sync_copy(nfd_v, nfd_out.at[pl.ds(base, _EPW)])


def _sc_gather(ef_rows, src, dst, nfm):
    mesh = plsc.VectorSubcoreMesh(core_axis_name="c", subcore_axis_name="s")
    f = pl.kernel(
        _sc_gather_body,
        mesh=mesh,
        out_type=[
            jax.ShapeDtypeStruct((E, 128), jnp.float32),
            jax.ShapeDtypeStruct((E, 128), jnp.float32),
            jax.ShapeDtypeStruct((E, 128), jnp.float32),
            jax.ShapeDtypeStruct((E, 128), jnp.float32),
            jax.ShapeDtypeStruct((E, D_IN), jnp.float32),
            jax.ShapeDtypeStruct((E, D_IN), jnp.float32),
        ],
        scratch_types=[
            pltpu.VMEM((_EPW,), jnp.int32),
            pltpu.VMEM((_EPW,), jnp.int32),
            pltpu.VMEM((_EPW,), jnp.int32),
            pltpu.VMEM((_EPW, 128), jnp.float32),
            pltpu.VMEM((_EPW, 128), jnp.float32),
            pltpu.VMEM((_EPW, 128), jnp.float32),
            pltpu.VMEM((_EPW, 128), jnp.float32),
            pltpu.VMEM((_EPW, D_IN), jnp.float32),
            pltpu.VMEM((_EPW, D_IN), jnp.float32),
            pltpu.SemaphoreType.DMA,
        ],
    )
    return f(ef_rows, src, dst, nfm)


BLK = 256
NBLK = E // BLK

_CD1 = (((1,), (1,)), ((), ()))


def _tc_body(consts, nfm, adj, src2, dst2, b0, b1, b2, b3, dstcol,
             nfs, nfd, wnT, wnb, weflat, web, atnq, atnk, atee, aten,
             qtnn, qtne, qtef, qtee, ladj,
             out, efl_s, p_s):
    i = pl.program_id(0)

    @pl.when(i == 0)
    def _small():
        atn_b = consts[0:1, 0:1]
        ate_b = consts[0:1, 1:2]
        qtn_b = consts[0:1, 2:3]

        nf = jnp.dot(nfm[...], wnT[...],
                     preferred_element_type=jnp.float32) + wnb[...]
        jj = lax.broadcasted_iota(jnp.int32, (E, 128), 1)
        sel = jj == (dstcol[...] & 127)
        ef_line = web[...]
        for k, bk in enumerate((b0, b1, b2, b3)):
            v_k = jnp.sum(jnp.where(sel, bk[...], 0.0),
                          axis=1, keepdims=True)
            ef_line = ef_line + v_k * weflat[0:1, k * E_OUT:(k + 1) * E_OUT]
        efl_s[...] = ef_line
        p_s[...] = lax.dot_general(qtef[...], ef_line, _CD1,
                                   preferred_element_type=jnp.float32)

        deg = jnp.sum(adj[...], axis=1, keepdims=True)
        qn = lax.dot_general(atnq[...], nf, _CD1,
                             preferred_element_type=jnp.float32)
        kn = lax.dot_general(nf, atnk[...], _CD1,
                             preferred_element_type=jnp.float32)
        s_nn = _lrelu(qn + kn + atn_b)
        colid = lax.broadcasted_iota(jnp.int32, (N, N), 1).astype(
            jnp.float32)
        attn_nn = _row_softmax(s_nn, colid < deg)
        out[0:N, 0:D_OUT] = jnp.maximum(
            jnp.dot(attn_nn, nf, preferred_element_type=jnp.float32), 0.0)

        a_row = lax.dot_general(atee[...], ef_line, _CD1,
                                preferred_element_type=jnp.float32)
        c_col = lax.dot_general(nf, aten[...], _CD1,
                                preferred_element_type=jnp.float32)
        srcr = src2[...]
        dstr = dst2[...]
        ch = 256
        for c in range(N // ch):
            ids = lax.broadcasted_iota(jnp.int32, (ch, 1), 0) + c * ch
            inc = (srcr == ids) | (dstr == ids)
            s_ne = _lrelu(a_row + c_col[c * ch:(c + 1) * ch, :] + ate_b)
            attn = _row_softmax(s_ne, inc)
            out[c * ch:(c + 1) * ch, D_OUT:D_OUT + E_OUT] = jnp.maximum(
                jnp.dot(attn, ef_line, preferred_element_type=jnp.float32),
                0.0)

        nfsrc = jnp.dot(nfs[...], wnT[...],
                        preferred_element_type=jnp.float32) + wnb[...]
        nfdst = jnp.dot(nfd[...], wnT[...],
                        preferred_element_type=jnp.float32) + wnb[...]
        sl_s = lax.dot_general(nfsrc, qtnn[...], _CD1,
                               preferred_element_type=jnp.float32)
        sl_d = lax.dot_general(nfdst, qtnn[...], _CD1,
                               preferred_element_type=jnp.float32)
        er = lax.dot_general(ef_line, qtne[...], _CD1,
                             preferred_element_type=jnp.float32)
        s0 = _lrelu(sl_s + er + qtn_b)
        s1 = _lrelu(sl_d + er + qtn_b)
        m = jnp.maximum(s0, s1)
        e0 = jnp.exp(s0 - m)
        e1 = jnp.exp(s1 - m)
        den = e0 + e1
        out[N:N + E, 0:D_OUT] = jnp.maximum(
            (e0 / den) * nfsrc + (e1 / den) * nfdst, 0.0)

    qte_b = consts[0:1, 3:4]
    e_full = efl_s[...]
    qv = lax.dot_general(efl_s[pl.ds(i * BLK, BLK), :], qtee[...], _CD1,
                         preferred_element_type=jnp.float32)
    s_ee = _lrelu(p_s[...] + qv + qte_b)
    attn = _row_softmax(s_ee, ladj[...] > 0)
    out[pl.ds(N + i * BLK, BLK), D_OUT:D_OUT + E_OUT] = jnp.maximum(
        jnp.dot(attn, e_full, preferred_element_type=jnp.float32), 0.0)


def _tc_fused(consts, nfm, adj, src2, dst2, b0, b1, b2, b3, dstcol, nfs, nfd,
              wnT, wnb, weflat, web, atnq, atnk, atee, aten, qtnn, qtne,
              qtef, qtee, ladj):
    full = lambda shape: pl.BlockSpec(shape, lambda i: (0,) * len(shape))
    return pl.pallas_call(
        _tc_body,
        grid=(NBLK,),
        in_specs=[
            full((1, 8)),
            full((N, D_IN)),
            full((N, N)),
            full((1, E)),
            full((1, E)),
            full((E, 128)),
            full((E, 128)),
            full((E, 128)),
            full((E, 128)),
            full((E, 1)),
            full((E, D_IN)),
            full((E, D_IN)),
            full((D_IN, D_OUT)),
            full((1, D_OUT)),
            full((1, E_IN * E_OUT)),
            full((1, E_OUT)),
            full((1, D_OUT)),
            full((1, D_OUT)),
            full((1, E_OUT)),
            full((1, D_OUT)),
            full((1, D_OUT)),
            full((1, E_OUT)),
            full((1, E_OUT)),
            full((1, E_OUT)),
            pl.BlockSpec((BLK, E), lambda i: (i, 0)),
        ],
        out_specs=full((N + E, D_OUT + E_OUT)),
        out_shape=jax.ShapeDtypeStruct((N + E, D_OUT + E_OUT), jnp.float32),
        scratch_shapes=[
            pltpu.VMEM((E, E_OUT), jnp.float32),
            pltpu.VMEM((1, E), jnp.float32),
        ],
    )(consts, nfm, adj, src2, dst2, b0, b1, b2, b3, dstcol, nfs, nfd,
      wnT, wnb, weflat, web, atnq, atnk, atee, aten, qtnn, qtne, qtef,
      qtee, ladj)


def kernel(node_features, edge_index, line_adj_matrix, adj_matrix,
           edge_features, Wn_w, Wn_b, We_w, We_b, atn_w, atn_b, ate_w, ate_b,
           qtn_w, qtn_b, qte_w, qte_b):
    src = edge_index[0].astype(jnp.int32)
    dst = edge_index[1].astype(jnp.int32)
    ef_rows = (edge_features.reshape(N, _NJB, 128, E_IN)
               .transpose(0, 1, 3, 2)
               .reshape(N * _NJB * E_IN, 128))

    b0, b1, b2, b3, nfs_raw, nfd_raw = _sc_gather(
        ef_rows, src, dst, node_features)

    consts = jnp.concatenate([
        atn_b.reshape(1), ate_b.reshape(1), qtn_b.reshape(1),
        qte_b.reshape(1), jnp.zeros((4,), jnp.float32)]).reshape(1, 8)

    return _tc_fused(
        consts, node_features, adj_matrix,
        src.reshape(1, E), dst.reshape(1, E),
        b0, b1, b2, b3, dst.reshape(E, 1), nfs_raw, nfd_raw,
        Wn_w.T, Wn_b.reshape(1, D_OUT),
        We_w.T.reshape(1, E_IN * E_OUT), We_b.reshape(1, E_OUT),
        atn_w[:, :D_OUT], atn_w[:, D_OUT:],
        ate_w[:, :E_OUT], ate_w[:, E_OUT:],
        qtn_w[:, :D_OUT], qtn_w[:, D_OUT:],
        qte_w[:, :E_OUT], qte_w[:, E_OUT:], line_adj_matrix)
